# Initial kernel scaffold; baseline (speedup 1.0000x reference)
import jax
import jax.numpy as jnp
from jax.experimental import pallas as pl


def kernel(x, W1, b1, W2, b2):
    raise NotImplementedError("write your pallas kernel here")


# speedup vs baseline: 1.2977x; 1.2977x over previous
import functools

import jax
import jax.numpy as jnp
from jax import lax
from jax.experimental import pallas as pl
from jax.experimental.pallas import tpu as pltpu
from jax.experimental.pallas import tpu_sc as plsc


_ERF_COEFFS = (7.85386146e-05, -0.000801019371, 0.00518832775, -0.0268538129,
               0.112835854, -0.37612626, 1.12837911)
_ERFC_P1 = (0.0232682, -0.138703942, 0.368742466, -0.582473278, 0.621000469,
            -0.494451523, 0.340488, -0.274112701, 0.563825965)
_ERFC_P2 = (-10.477664, 12.9772, -7.49551868, 2.92101908, -1.01526523,
            0.42184633, -0.282076746, 0.564189494)


def _erfc_f32(z):
    one = jnp.float32(1.0)
    x2 = z * z
    p = jnp.full_like(z, _ERF_COEFFS[0])
    for c in _ERF_COEFFS[1:]:
        p = p * x2 + jnp.float32(c)
    erf = z * p
    r_small = one - erf
    az = jnp.abs(z)
    q = one / x2
    r1 = q * jnp.float32(_ERFC_P1[0])
    for c in _ERFC_P1[1:-1]:
        r1 = (r1 + jnp.float32(c)) * q
    r1 = r1 + jnp.float32(_ERFC_P1[-1])
    r2 = q * jnp.float32(_ERFC_P2[0])
    for c in _ERFC_P2[1:-1]:
        r2 = (r2 + jnp.float32(c)) * q
    r2 = r2 + jnp.float32(_ERFC_P2[-1])
    poly = jnp.where(az < jnp.float32(2.0), r1, r2)
    base = jnp.exp(-x2) * (one / az)
    val = base * poly
    val = jnp.where(-x2 < jnp.float32(-88.7228394), jnp.float32(0.0), val)
    val = jnp.where(z < jnp.float32(0.0), jnp.float32(2.0) - val, val)
    return jnp.where(az < one, r_small, val)


def _gelu_exact(h):
    z = (-h) * jnp.float32(0.707106769)
    return (h * jnp.float32(0.5)) * _erfc_f32(z)


def _score_body(x_ref, w1_ref, b1_ref, w2_ref, b2_ref, s_ref):
    h = jnp.dot(x_ref[...], w1_ref[...], preferred_element_type=jnp.float32)
    h = h + b1_ref[...]
    h = _gelu_exact(h)
    s = jnp.dot(h, w2_ref[...], preferred_element_type=jnp.float32)
    s_ref[...] = s + b2_ref[...]


def _scores(xf, W1, b1, W2, b2):
    M, D = xf.shape
    H = W1.shape[1]
    BLK = 1024
    grid = (M // BLK,)
    return pl.pallas_call(
        _score_body,
        grid=grid,
        in_specs=[
            pl.BlockSpec((BLK, D), lambda i: (i, 0)),
            pl.BlockSpec((D, H), lambda i: (0, 0)),
            pl.BlockSpec((1, H), lambda i: (0, 0)),
            pl.BlockSpec((H, 1), lambda i: (0, 0)),
            pl.BlockSpec((1, 1), lambda i: (0, 0)),
        ],
        out_specs=pl.BlockSpec((BLK, 1), lambda i: (i, 0)),
        out_shape=jax.ShapeDtypeStruct((M, 1), jnp.float32),
    )(xf, W1, b1.reshape(1, H), W2, b2.reshape(1, 1))


def _sort_body(s_ref, idx_ref, fidx_ref):
    B, N = s_ref.shape
    K = N // 2
    s = s_ref[...]
    idx = lax.broadcasted_iota(jnp.int32, (B, N), 1)
    pos = lax.broadcasted_iota(jnp.int32, (B, N), 1)

    k = 2
    while k <= N:
        j = k // 2
        while j >= 1:
            s_m = pltpu.roll(s, j, 1)
            s_p = pltpu.roll(s, N - j, 1)
            i_m = pltpu.roll(idx, j, 1)
            i_p = pltpu.roll(idx, N - j, 1)
            hi = (pos & j) != 0
            s_part = jnp.where(hi, s_m, s_p)
            i_part = jnp.where(hi, i_m, i_p)
            first = (s > s_part) | ((s == s_part) & (idx < i_part))
            dirf = (pos & k) == 0
            want_first = jnp.logical_not(hi) == dirf
            take = jnp.logical_xor(first, want_first)
            s = jnp.where(take, s_part, s)
            idx = jnp.where(take, i_part, idx)
            j //= 2
        k *= 2

    top = idx[:, :K]
    idx_ref[...] = top
    off = lax.broadcasted_iota(jnp.int32, (B, K), 0) * N
    fidx_ref[...] = top + off


def _topk(scores):
    B, N = scores.shape
    K = N // 2
    return pl.pallas_call(
        _sort_body,
        out_shape=(
            jax.ShapeDtypeStruct((B, K), jnp.int32),
            jax.ShapeDtypeStruct((B, K), jnp.int32),
        ),
    )(scores)


_INFO = plsc.get_sparse_core_info()
_NC = _INFO.num_cores
_NS = _INFO.num_subcores
_NW = _NC * _NS


def _make_gather(R, D, CHUNK):
    rpw = R // _NW
    n_chunks = rpw // CHUNK
    mesh = plsc.VectorSubcoreMesh(core_axis_name="c", subcore_axis_name="s")

    @functools.partial(
        pl.kernel,
        mesh=mesh,
        out_type=jax.ShapeDtypeStruct((R, D), jnp.float32),
        scratch_types=[
            pltpu.VMEM((rpw,), jnp.int32),
            pltpu.VMEM((CHUNK, D), jnp.float32),
            pltpu.VMEM((CHUNK, D), jnp.float32),
            pltpu.SemaphoreType.DMA,
            pltpu.SemaphoreType.DMA,
        ],
    )
    def gather(x_hbm, fidx_hbm, out_hbm, idx_v, buf0, buf1, sem0, sem1):
        wid = lax.axis_index("s") * _NC + lax.axis_index("c")
        base = wid * rpw
        pltpu.sync_copy(fidx_hbm.at[pl.ds(base, rpw)], idx_v)

        bufs = (buf0, buf1)
        sems = (sem0, sem1)

        def start(g):
            return pltpu.async_copy(
                x_hbm.at[idx_v.at[pl.ds(g * CHUNK, CHUNK)]],
                bufs[g % 2], sems[g % 2])

        pend = start(0)
        for g in range(n_chunks):
            nxt = start(g + 1) if g + 1 < n_chunks else None
            pend.wait()
            pltpu.sync_copy(bufs[g % 2], out_hbm.at[pl.ds(base + g * CHUNK, CHUNK)])
            pend = nxt

    return gather


def kernel(x, W1, b1, W2, b2):
    B, N, D = x.shape
    K = N // 2
    xf = x.reshape(B * N, D)
    s = _scores(xf, W1, b1, W2, b2).reshape(B, N)
    idx, fidx = _topk(s)
    out = _make_gather(B * K, D, 32)(xf, fidx.reshape(B * K))
    return out.reshape(B, K, D), idx
